# Initial kernel scaffold; baseline (speedup 1.0000x reference)
import jax
import jax.numpy as jnp
from jax.experimental import pallas as pl


def kernel(pos, batch, kp1_pts, W1, kp2_pts, W2, W_fp2, b_fp2, W_fp1a, b_fp1a, W_fp1b, b_fp1b, Wc1, bc1, Wc2, bc2):
    raise NotImplementedError("write your pallas kernel here")


# speedup vs baseline: 3.5664x; 3.5664x over previous
import functools

import jax
import jax.numpy as jnp
from jax import lax
from jax.experimental import pallas as pl
from jax.experimental.pallas import tpu as pltpu
from jax.experimental.pallas import tpu_sc as plsc

N_POINTS = 8192
NBR = 32
KK = 15
BIG = 1e9
PAD_COORD = 1e4


def _topk_kernel(q_ref, ct_ref, idx_ref, val_ref, *, kout, ncp):
    qt = q_ref[...]
    ct = ct_ref[...]
    dx = qt[:, 0:1] - ct[0:1, :]
    dy = qt[:, 1:2] - ct[1:2, :]
    dz = qt[:, 2:3] - ct[2:3, :]
    d2 = (dx * dx + dy * dy) + dz * dz
    d2 = d2 + jnp.where(qt[:, 3:4] != ct[3:4, :], jnp.float32(1e6),
                        jnp.float32(0.0))
    lane = lax.broadcasted_iota(jnp.int32, d2.shape, 1)
    for i in range(kout):
        m = jnp.min(d2, axis=1, keepdims=True)
        sel = jnp.where(d2 == m, lane, jnp.int32(2 * ncp))
        amin = jnp.min(sel, axis=1, keepdims=True)
        val_ref[:, i:i + 1] = m
        idx_ref[:, i:i + 1] = amin
        d2 = jnp.where(lane == amin, jnp.float32(BIG * 10.0), d2)


def _topk_select(qp, cp, kout, tq=128):
    mqp = qp.shape[0]
    ncp = cp.shape[0]
    ct = cp.T
    grid = (mqp // tq,)
    return pl.pallas_call(
        functools.partial(_topk_kernel, kout=kout, ncp=ncp),
        grid=grid,
        in_specs=[
            pl.BlockSpec((tq, 8), lambda i: (i, 0)),
            pl.BlockSpec((8, ncp), lambda i: (0, 0)),
        ],
        out_specs=[
            pl.BlockSpec((tq, kout), lambda i: (i, 0)),
            pl.BlockSpec((tq, kout), lambda i: (i, 0)),
        ],
        out_shape=[
            jax.ShapeDtypeStruct((mqp, kout), jnp.int32),
            jax.ShapeDtypeStruct((mqp, kout), jnp.float32),
        ],
    )(qp, ct)


def _sc_gather(table, idx):
    info = plsc.get_sparse_core_info()
    nc, ns = info.num_cores, info.num_subcores
    nw = nc * ns
    b = idx.shape[0]
    d = table.shape[1]
    bpw = b // nw
    chunk = 8
    for c in range(min(bpw, 256), 7, -8):
        if bpw % c == 0:
            chunk = c
            break
    nch = bpw // chunk
    mesh = plsc.VectorSubcoreMesh(core_axis_name="c", subcore_axis_name="s")

    @functools.partial(
        pl.kernel,
        mesh=mesh,
        out_type=jax.ShapeDtypeStruct((b, d), jnp.float32),
        scratch_types=[
            pltpu.VMEM((chunk,), jnp.int32),
            pltpu.VMEM((chunk, d), jnp.float32),
            pltpu.SemaphoreType.DMA,
        ],
    )
    def gk(table_hbm, idx_hbm, out_hbm, idx_v, rows_v, sem):
        wid = lax.axis_index("s") * nc + lax.axis_index("c")
        base = wid * bpw
        for j in range(nch):
            off = base + j * chunk
            pltpu.sync_copy(idx_hbm.at[pl.ds(off, chunk)], idx_v)
            pltpu.async_copy(table_hbm.at[idx_v], rows_v, sem).wait()
            pltpu.sync_copy(rows_v, out_hbm.at[pl.ds(off, chunk)])

    return gk(table, idx)


def _kpconv_kernel(q_ref, d2n_ref, g_ref, kp_ref, w_ref, out_ref, *,
                   r, cin, xoff, tq):
    g = g_ref[...].reshape(tq, NBR, g_ref.shape[1])
    pos_j = g[:, :, 0:3]
    x_j = g[:, :, xoff:xoff + cin]
    qpos = q_ref[...][:, None, 0:3]
    y = pos_j - qpos
    mask = (d2n_ref[...] <= r * r).astype(jnp.float32)
    xn = x_j * mask[:, :, None]
    fs = []
    for k in range(KK):
        kv = kp_ref[k:k + 1, 0:3].reshape(1, 1, 3)
        diff = y - kv
        dist = jnp.sqrt(jnp.sum(diff * diff, axis=2) + 1e-12)
        h = jnp.maximum(0.0, 1.0 - dist / r)
        fs.append(jnp.sum(h[:, :, None] * xn, axis=1))
    f = jnp.concatenate(fs, axis=1)
    kcp = w_ref.shape[0]
    if f.shape[1] < kcp:
        f = jnp.concatenate(
            [f, jnp.zeros((tq, kcp - f.shape[1]), jnp.float32)], axis=1)
    out_ref[...] = jnp.dot(f, w_ref[...], preferred_element_type=jnp.float32)


def _kpconv(qp, d2n, rows, kpts, w_flat, r, cin, xoff, cout, tq=128):
    mqp = qp.shape[0]
    dt = rows.shape[1]
    kcp = w_flat.shape[0]
    grid = (mqp // tq,)
    return pl.pallas_call(
        functools.partial(_kpconv_kernel, r=r, cin=cin, xoff=xoff, tq=tq),
        grid=grid,
        in_specs=[
            pl.BlockSpec((tq, 8), lambda i: (i, 0)),
            pl.BlockSpec((tq, NBR), lambda i: (i, 0)),
            pl.BlockSpec((tq * NBR, dt), lambda i: (i, 0)),
            pl.BlockSpec((16, 8), lambda i: (0, 0)),
            pl.BlockSpec((kcp, cout), lambda i: (0, 0)),
        ],
        out_specs=pl.BlockSpec((tq, cout), lambda i: (i, 0)),
        out_shape=jax.ShapeDtypeStruct((mqp, cout), jnp.float32),
    )(qp, d2n, rows, kpts, w_flat)


def _interp_weights(d2n):
    d = jnp.maximum(d2n, 1e-16)
    w = 1.0 / d
    return w / jnp.sum(w, axis=1, keepdims=True)


def _fp2_kernel(d2n_ref, g_ref, x1_ref, w_ref, b_ref, out_ref, *, tq, c):
    g = g_ref[...].reshape(tq, 3, g_ref.shape[1])[:, :, :c]
    w3 = _interp_weights(d2n_ref[...])
    xi = jnp.sum(g * w3[:, :, None], axis=1)
    cat = jnp.concatenate([xi, x1_ref[...]], axis=1)
    out_ref[...] = (
        jnp.dot(cat, w_ref[...], preferred_element_type=jnp.float32)
        + b_ref[...]
    )


def _fp1_cls_kernel(d2n_ref, g_ref, wa_ref, ba_ref, wb_ref, bb_ref,
                    wc1_ref, bc1_ref, wc2_ref, bc2_ref, out_ref, *, tq, c):
    g = g_ref[...].reshape(tq, 3, g_ref.shape[1])[:, :, :c]
    w3 = _interp_weights(d2n_ref[...])
    xj = jnp.sum(g * w3[:, :, None], axis=1)
    xj = jnp.maximum(
        jnp.dot(xj, wa_ref[...], preferred_element_type=jnp.float32)
        + ba_ref[...], 0.0)
    xj = (jnp.dot(xj, wb_ref[...], preferred_element_type=jnp.float32)
          + bb_ref[...])
    xc = jnp.maximum(
        jnp.dot(xj, wc1_ref[...], preferred_element_type=jnp.float32)
        + bc1_ref[...], 0.0)
    xc = (jnp.dot(xc, wc2_ref[...], preferred_element_type=jnp.float32)
          + bc2_ref[...])
    mx = jnp.max(xc, axis=1, keepdims=True)
    z = xc - mx
    lse = jnp.log(jnp.sum(jnp.exp(z), axis=1, keepdims=True))
    out_ref[...] = z - lse


def _pad_rows(a, n):
    return jnp.pad(a, ((0, n - a.shape[0]), (0, 0)))


def _pack(posn, batchf, pad_to):
    m = posn.shape[0]
    p = jnp.concatenate(
        [posn, batchf[:, None], jnp.zeros((m, 4), jnp.float32)], axis=1)
    if pad_to > m:
        pad = jnp.zeros((pad_to - m, 8), jnp.float32).at[:, 0].set(PAD_COORD)
        p = jnp.concatenate([p, pad], axis=0)
    return p


def kernel(pos, batch, kp1_pts, W1, kp2_pts, W2, W_fp2, b_fp2, W_fp1a,
           b_fp1a, W_fp1b, b_fp1b, Wc1, bc1, Wc2, bc2):
    mx = jnp.max(pos)
    mn = jnp.min(pos)
    posn = (pos - (mx + mn) / 2.0) / jnp.abs(mx - mn)
    bf = batch.astype(jnp.float32)

    m1, m1p = (N_POINTS + 2) // 3, 2816
    m2, m2p = (m1 + 1) // 2, 1408

    p0 = _pack(posn, bf, N_POINTS)
    q1 = _pack(posn[::3], bf[::3], m1p)
    q2 = _pack(posn[::3][::2], bf[::3][::2], m2p)

    idx1, d2n1 = _topk_select(q1, p0, NBR)
    idx2, d2n2 = _topk_select(q2, q1, NBR)
    idxi2, d2ni2 = _topk_select(q1, q2, 3)
    idxi1, d2ni1 = _topk_select(p0, q1, 3)

    t1 = jnp.pad(posn, ((0, 0), (0, 125)))
    g1 = _sc_gather(t1, idx1.reshape(-1))
    kp1 = jnp.pad(kp1_pts, ((0, 1), (0, 5)))
    w1f = jnp.pad(W1.reshape(KK * 3, 32), ((0, 3), (0, 0)))
    x1 = _kpconv(q1, d2n1, g1, kp1, w1f, 0.2, 3, 0, 32)

    t2 = jnp.concatenate(
        [q1[:, 0:3], x1, jnp.zeros((m1p, 93), jnp.float32)], axis=1)
    g2 = _sc_gather(t2, idx2.reshape(-1))
    kp2 = jnp.pad(kp2_pts, ((0, 1), (0, 5)))
    w2f = W2.reshape(KK * 32, 64)
    x2 = _kpconv(q2, d2n2, g2, kp2, w2f, 0.4, 32, 3, 64)

    gi2 = _sc_gather(jnp.pad(x2, ((0, 0), (0, 64))),
                     idxi2.reshape(-1))
    tq = 128
    xf = pl.pallas_call(
        functools.partial(_fp2_kernel, tq=tq, c=64),
        grid=(m1p // tq,),
        in_specs=[
            pl.BlockSpec((tq, 3), lambda i: (i, 0)),
            pl.BlockSpec((tq * 3, 128), lambda i: (i, 0)),
            pl.BlockSpec((tq, 32), lambda i: (i, 0)),
            pl.BlockSpec((96, 32), lambda i: (0, 0)),
            pl.BlockSpec((1, 32), lambda i: (0, 0)),
        ],
        out_specs=pl.BlockSpec((tq, 32), lambda i: (i, 0)),
        out_shape=jax.ShapeDtypeStruct((m1p, 32), jnp.float32),
    )(d2ni2, gi2, x1, W_fp2, b_fp2[None, :])

    gi1 = _sc_gather(jnp.pad(xf, ((0, 0), (0, 96))),
                     idxi1.reshape(-1))
    out = pl.pallas_call(
        functools.partial(_fp1_cls_kernel, tq=tq, c=32),
        grid=(N_POINTS // tq,),
        in_specs=[
            pl.BlockSpec((tq, 3), lambda i: (i, 0)),
            pl.BlockSpec((tq * 3, 128), lambda i: (i, 0)),
            pl.BlockSpec((32, 32), lambda i: (0, 0)),
            pl.BlockSpec((1, 32), lambda i: (0, 0)),
            pl.BlockSpec((32, 32), lambda i: (0, 0)),
            pl.BlockSpec((1, 32), lambda i: (0, 0)),
            pl.BlockSpec((32, 16), lambda i: (0, 0)),
            pl.BlockSpec((1, 16), lambda i: (0, 0)),
            pl.BlockSpec((16, 50), lambda i: (0, 0)),
            pl.BlockSpec((1, 50), lambda i: (0, 0)),
        ],
        out_specs=pl.BlockSpec((tq, 50), lambda i: (i, 0)),
        out_shape=jax.ShapeDtypeStruct((N_POINTS, 50), jnp.float32),
    )(d2ni1, gi1, W_fp1a, b_fp1a[None, :], W_fp1b, b_fp1b[None, :],
      Wc1, bc1[None, :], Wc2, bc2[None, :])
    return out


# speedup vs baseline: 4.1547x; 1.1650x over previous
import functools

import jax
import jax.numpy as jnp
from jax import lax
from jax.experimental import pallas as pl
from jax.experimental.pallas import tpu as pltpu
from jax.experimental.pallas import tpu_sc as plsc

N_POINTS = 8192
NBR = 32
KK = 15
BIG = 1e9
PAD_COORD = 1e4


def _topk_kernel(s_ref, q_ref, cw_ref, idx_ref, val_ref, *, kout, w):
    start = s_ref[pl.program_id(0)]
    qt = q_ref[...]
    ct = cw_ref[...].reshape(8, w)
    dx = qt[:, 0:1] - ct[0:1, :]
    dy = qt[:, 1:2] - ct[1:2, :]
    dz = qt[:, 2:3] - ct[2:3, :]
    d2 = (dx * dx + dy * dy) + dz * dz
    d2 = d2 + jnp.where(qt[:, 3:4] != ct[3:4, :], jnp.float32(1e6),
                        jnp.float32(0.0))
    lane = lax.broadcasted_iota(jnp.int32, d2.shape, 1)
    for i in range(kout):
        m = jnp.min(d2, axis=1, keepdims=True)
        sel = jnp.where(d2 == m, lane, jnp.int32(1 << 30))
        amin = jnp.min(sel, axis=1, keepdims=True)
        val_ref[:, i:i + 1] = m
        idx_ref[:, i:i + 1] = amin + start
        d2 = jnp.where(sel == amin, jnp.float32(BIG * 10.0), d2)


def _topk_select(qp, cp, kout, starts, w, tq=128):
    mqp = qp.shape[0]
    ct = cp.T
    ntiles = mqp // tq
    windows = jax.vmap(
        lambda s: lax.dynamic_slice(ct, (0, s), (8, w)))(starts)
    grid_spec = pltpu.PrefetchScalarGridSpec(
        num_scalar_prefetch=1,
        grid=(ntiles,),
        in_specs=[
            pl.BlockSpec((tq, 8), lambda i, s: (i, 0)),
            pl.BlockSpec((1, 8, w), lambda i, s: (i, 0, 0)),
        ],
        out_specs=[
            pl.BlockSpec((tq, kout), lambda i, s: (i, 0)),
            pl.BlockSpec((tq, kout), lambda i, s: (i, 0)),
        ],
    )
    return pl.pallas_call(
        functools.partial(_topk_kernel, kout=kout, w=w),
        grid_spec=grid_spec,
        out_shape=[
            jax.ShapeDtypeStruct((mqp, kout), jnp.int32),
            jax.ShapeDtypeStruct((mqp, kout), jnp.float32),
        ],
    )(starts, qp, windows)


def _windowed_select(qp, cp, bc, kout, w, tq=128):
    mqp = qp.shape[0]
    ncp = cp.shape[0]
    ntiles = mqp // tq
    grid4 = jnp.arange(4, dtype=jnp.float32)
    cstart = jnp.searchsorted(bc, grid4).astype(jnp.int32)
    cend = jnp.searchsorted(bc, grid4 + 0.5).astype(jnp.int32)
    tidx = jnp.arange(ntiles, dtype=jnp.int32)
    blo = qp[tidx * tq, 3].astype(jnp.int32)
    bhi = jnp.maximum(blo, qp[tidx * tq + tq - 1, 3].astype(jnp.int32))
    blo = jnp.clip(blo, 0, 3)
    bhi = jnp.clip(bhi, 0, 3)
    starts = jnp.clip((cstart[blo] // 128) * 128, 0, ncp - w)
    ok = jnp.all(cend[bhi] <= starts + w)
    zeros = jnp.zeros((ntiles,), jnp.int32)
    return lax.cond(
        ok,
        lambda: _topk_select(qp, cp, kout, starts, w, tq),
        lambda: _topk_select(qp, cp, kout, zeros, ncp, tq),
    )


def _sc_gather(table, idx):
    info = plsc.get_sparse_core_info()
    nc, ns = info.num_cores, info.num_subcores
    nw = nc * ns
    b = idx.shape[0]
    d = table.shape[1]
    bpw = b // nw
    chunk = 8
    for c in range(min(bpw, 256), 7, -8):
        if bpw % c == 0:
            chunk = c
            break
    nch = bpw // chunk
    mesh = plsc.VectorSubcoreMesh(core_axis_name="c", subcore_axis_name="s")

    @functools.partial(
        pl.kernel,
        mesh=mesh,
        out_type=jax.ShapeDtypeStruct((b, d), jnp.float32),
        scratch_types=[
            pltpu.VMEM((chunk,), jnp.int32),
            pltpu.VMEM((chunk, d), jnp.float32),
            pltpu.SemaphoreType.DMA,
        ],
    )
    def gk(table_hbm, idx_hbm, out_hbm, idx_v, rows_v, sem):
        wid = lax.axis_index("s") * nc + lax.axis_index("c")
        base = wid * bpw
        for j in range(nch):
            off = base + j * chunk
            pltpu.sync_copy(idx_hbm.at[pl.ds(off, chunk)], idx_v)
            pltpu.async_copy(table_hbm.at[idx_v], rows_v, sem).wait()
            pltpu.sync_copy(rows_v, out_hbm.at[pl.ds(off, chunk)])

    return gk(table, idx)


def _kpconv_kernel(q_ref, d2n_ref, g_ref, kp_ref, w_ref, out_ref, *,
                   r, cin, xoff, tq):
    g = g_ref[...].reshape(tq, NBR, g_ref.shape[1])
    pos_j = g[:, :, 0:3]
    x_j = g[:, :, xoff:xoff + cin]
    qpos = q_ref[...][:, None, 0:3]
    y = pos_j - qpos
    mask = (d2n_ref[...] <= r * r).astype(jnp.float32)
    xn = x_j * mask[:, :, None]
    fs = []
    for k in range(KK):
        kv = kp_ref[k:k + 1, 0:3].reshape(1, 1, 3)
        diff = y - kv
        dist = jnp.sqrt(jnp.sum(diff * diff, axis=2) + 1e-12)
        h = jnp.maximum(0.0, 1.0 - dist / r)
        fs.append(jnp.sum(h[:, :, None] * xn, axis=1))
    f = jnp.concatenate(fs, axis=1)
    kcp = w_ref.shape[0]
    if f.shape[1] < kcp:
        f = jnp.concatenate(
            [f, jnp.zeros((tq, kcp - f.shape[1]), jnp.float32)], axis=1)
    out_ref[...] = jnp.dot(f, w_ref[...], preferred_element_type=jnp.float32)


def _kpconv(qp, d2n, rows, kpts, w_flat, r, cin, xoff, cout, tq=128):
    mqp = qp.shape[0]
    dt = rows.shape[1]
    kcp = w_flat.shape[0]
    grid = (mqp // tq,)
    return pl.pallas_call(
        functools.partial(_kpconv_kernel, r=r, cin=cin, xoff=xoff, tq=tq),
        grid=grid,
        in_specs=[
            pl.BlockSpec((tq, 8), lambda i: (i, 0)),
            pl.BlockSpec((tq, NBR), lambda i: (i, 0)),
            pl.BlockSpec((tq * NBR, dt), lambda i: (i, 0)),
            pl.BlockSpec((16, 8), lambda i: (0, 0)),
            pl.BlockSpec((kcp, cout), lambda i: (0, 0)),
        ],
        out_specs=pl.BlockSpec((tq, cout), lambda i: (i, 0)),
        out_shape=jax.ShapeDtypeStruct((mqp, cout), jnp.float32),
    )(qp, d2n, rows, kpts, w_flat)


def _interp_weights(d2n):
    d = jnp.maximum(d2n, 1e-16)
    w = 1.0 / d
    return w / jnp.sum(w, axis=1, keepdims=True)


def _fp2_kernel(d2n_ref, g_ref, x1_ref, w_ref, b_ref, out_ref, *, tq, c):
    g = g_ref[...].reshape(tq, 3, g_ref.shape[1])[:, :, :c]
    w3 = _interp_weights(d2n_ref[...])
    xi = jnp.sum(g * w3[:, :, None], axis=1)
    cat = jnp.concatenate([xi, x1_ref[...]], axis=1)
    out_ref[...] = (
        jnp.dot(cat, w_ref[...], preferred_element_type=jnp.float32)
        + b_ref[...]
    )


def _fp1_cls_kernel(d2n_ref, g_ref, wa_ref, ba_ref, wb_ref, bb_ref,
                    wc1_ref, bc1_ref, wc2_ref, bc2_ref, out_ref, *, tq, c):
    g = g_ref[...].reshape(tq, 3, g_ref.shape[1])[:, :, :c]
    w3 = _interp_weights(d2n_ref[...])
    xj = jnp.sum(g * w3[:, :, None], axis=1)
    xj = jnp.maximum(
        jnp.dot(xj, wa_ref[...], preferred_element_type=jnp.float32)
        + ba_ref[...], 0.0)
    xj = (jnp.dot(xj, wb_ref[...], preferred_element_type=jnp.float32)
          + bb_ref[...])
    xc = jnp.maximum(
        jnp.dot(xj, wc1_ref[...], preferred_element_type=jnp.float32)
        + bc1_ref[...], 0.0)
    xc = (jnp.dot(xc, wc2_ref[...], preferred_element_type=jnp.float32)
          + bc2_ref[...])
    mx = jnp.max(xc, axis=1, keepdims=True)
    z = xc - mx
    lse = jnp.log(jnp.sum(jnp.exp(z), axis=1, keepdims=True))
    out_ref[...] = z - lse


def _pad_rows(a, n):
    return jnp.pad(a, ((0, n - a.shape[0]), (0, 0)))


def _pack(posn, batchf, pad_to):
    m = posn.shape[0]
    p = jnp.concatenate(
        [posn, batchf[:, None], jnp.zeros((m, 4), jnp.float32)], axis=1)
    if pad_to > m:
        pad = jnp.zeros((pad_to - m, 8), jnp.float32).at[:, 0].set(PAD_COORD)
        p = jnp.concatenate([p, pad], axis=0)
    return p


def kernel(pos, batch, kp1_pts, W1, kp2_pts, W2, W_fp2, b_fp2, W_fp1a,
           b_fp1a, W_fp1b, b_fp1b, Wc1, bc1, Wc2, bc2):
    mx = jnp.max(pos)
    mn = jnp.min(pos)
    posn = (pos - (mx + mn) / 2.0) / jnp.abs(mx - mn)
    bf = batch.astype(jnp.float32)

    m1, m1p = (N_POINTS + 2) // 3, 2816
    m2, m2p = (m1 + 1) // 2, 1408

    p0 = _pack(posn, bf, N_POINTS)
    q1 = _pack(posn[::3], bf[::3], m1p)
    q2 = _pack(posn[::3][::2], bf[::3][::2], m2p)

    b0 = bf
    b1 = bf[::3]
    b2 = bf[::3][::2]
    idx1, d2n1 = _windowed_select(q1, p0, b0, NBR, 4608)
    idx2, d2n2 = _windowed_select(q2, q1, b1, NBR, 1664)
    idxi2, d2ni2 = _windowed_select(q1, q2, b2, 3, 896)
    idxi1, d2ni1 = _windowed_select(p0, q1, b1, 3, 1664)

    t1 = jnp.pad(posn, ((0, 0), (0, 125)))
    g1 = _sc_gather(t1, idx1.reshape(-1))
    kp1 = jnp.pad(kp1_pts, ((0, 1), (0, 5)))
    w1f = jnp.pad(W1.reshape(KK * 3, 32), ((0, 3), (0, 0)))
    x1 = _kpconv(q1, d2n1, g1, kp1, w1f, 0.2, 3, 0, 32)

    t2 = jnp.concatenate(
        [q1[:, 0:3], x1, jnp.zeros((m1p, 93), jnp.float32)], axis=1)
    g2 = _sc_gather(t2, idx2.reshape(-1))
    kp2 = jnp.pad(kp2_pts, ((0, 1), (0, 5)))
    w2f = W2.reshape(KK * 32, 64)
    x2 = _kpconv(q2, d2n2, g2, kp2, w2f, 0.4, 32, 3, 64)

    gi2 = _sc_gather(jnp.pad(x2, ((0, 0), (0, 64))),
                     idxi2.reshape(-1))
    tq = 128
    xf = pl.pallas_call(
        functools.partial(_fp2_kernel, tq=tq, c=64),
        grid=(m1p // tq,),
        in_specs=[
            pl.BlockSpec((tq, 3), lambda i: (i, 0)),
            pl.BlockSpec((tq * 3, 128), lambda i: (i, 0)),
            pl.BlockSpec((tq, 32), lambda i: (i, 0)),
            pl.BlockSpec((96, 32), lambda i: (0, 0)),
            pl.BlockSpec((1, 32), lambda i: (0, 0)),
        ],
        out_specs=pl.BlockSpec((tq, 32), lambda i: (i, 0)),
        out_shape=jax.ShapeDtypeStruct((m1p, 32), jnp.float32),
    )(d2ni2, gi2, x1, W_fp2, b_fp2[None, :])

    gi1 = _sc_gather(jnp.pad(xf, ((0, 0), (0, 96))),
                     idxi1.reshape(-1))
    out = pl.pallas_call(
        functools.partial(_fp1_cls_kernel, tq=tq, c=32),
        grid=(N_POINTS // tq,),
        in_specs=[
            pl.BlockSpec((tq, 3), lambda i: (i, 0)),
            pl.BlockSpec((tq * 3, 128), lambda i: (i, 0)),
            pl.BlockSpec((32, 32), lambda i: (0, 0)),
            pl.BlockSpec((1, 32), lambda i: (0, 0)),
            pl.BlockSpec((32, 32), lambda i: (0, 0)),
            pl.BlockSpec((1, 32), lambda i: (0, 0)),
            pl.BlockSpec((32, 16), lambda i: (0, 0)),
            pl.BlockSpec((1, 16), lambda i: (0, 0)),
            pl.BlockSpec((16, 50), lambda i: (0, 0)),
            pl.BlockSpec((1, 50), lambda i: (0, 0)),
        ],
        out_specs=pl.BlockSpec((tq, 50), lambda i: (i, 0)),
        out_shape=jax.ShapeDtypeStruct((N_POINTS, 50), jnp.float32),
    )(d2ni1, gi1, W_fp1a, b_fp1a[None, :], W_fp1b, b_fp1b[None, :],
      Wc1, bc1[None, :], Wc2, bc2[None, :])
    return out


# speedup vs baseline: 4.3094x; 1.0372x over previous
import functools

import jax
import jax.numpy as jnp
from jax import lax
from jax.experimental import pallas as pl
from jax.experimental.pallas import tpu as pltpu
from jax.experimental.pallas import tpu_sc as plsc

N_POINTS = 8192
NBR = 32
KK = 15
BIG = 1e9
PAD_COORD = 1e4


def _topk_kernel(s_ref, q_ref, cw_ref, idx_ref, val_ref, *, kout, w):
    start = s_ref[pl.program_id(0)]
    qt = q_ref[...]
    ct = cw_ref[...].reshape(8, w)
    dx = qt[:, 0:1] - ct[0:1, :]
    dy = qt[:, 1:2] - ct[1:2, :]
    dz = qt[:, 2:3] - ct[2:3, :]
    d2 = (dx * dx + dy * dy) + dz * dz
    d2 = d2 + jnp.where(qt[:, 3:4] != ct[3:4, :], jnp.float32(1e6),
                        jnp.float32(0.0))
    lane = lax.broadcasted_iota(jnp.int32, d2.shape, 1)
    for i in range(kout):
        m = jnp.min(d2, axis=1, keepdims=True)
        sel = jnp.where(d2 == m, lane, jnp.int32(1 << 30))
        amin = jnp.min(sel, axis=1, keepdims=True)
        val_ref[:, i:i + 1] = m
        idx_ref[:, i:i + 1] = amin + start
        d2 = jnp.where(sel == amin, jnp.float32(BIG * 10.0), d2)


def _topk_select(qp, cp, kout, starts, w, tq=128):
    mqp = qp.shape[0]
    ct = cp.T
    ntiles = mqp // tq
    windows = jax.vmap(
        lambda s: lax.dynamic_slice(ct, (0, s), (8, w)))(starts)
    grid_spec = pltpu.PrefetchScalarGridSpec(
        num_scalar_prefetch=1,
        grid=(ntiles,),
        in_specs=[
            pl.BlockSpec((tq, 8), lambda i, s: (i, 0)),
            pl.BlockSpec((1, 8, w), lambda i, s: (i, 0, 0)),
        ],
        out_specs=[
            pl.BlockSpec((tq, kout), lambda i, s: (i, 0)),
            pl.BlockSpec((tq, kout), lambda i, s: (i, 0)),
        ],
    )
    return pl.pallas_call(
        functools.partial(_topk_kernel, kout=kout, w=w),
        grid_spec=grid_spec,
        out_shape=[
            jax.ShapeDtypeStruct((mqp, kout), jnp.int32),
            jax.ShapeDtypeStruct((mqp, kout), jnp.float32),
        ],
    )(starts, qp, windows)


def _windowed_select(qp, cp, bc, kout, w, tq=256):
    mqp = qp.shape[0]
    ncp = cp.shape[0]
    ntiles = mqp // tq
    grid4 = jnp.arange(4, dtype=jnp.float32)
    cstart = jnp.searchsorted(bc, grid4).astype(jnp.int32)
    cend = jnp.searchsorted(bc, grid4 + 0.5).astype(jnp.int32)
    tidx = jnp.arange(ntiles, dtype=jnp.int32)
    blo = qp[tidx * tq, 3].astype(jnp.int32)
    bhi = jnp.maximum(blo, qp[tidx * tq + tq - 1, 3].astype(jnp.int32))
    blo = jnp.clip(blo, 0, 3)
    bhi = jnp.clip(bhi, 0, 3)
    starts = jnp.clip((cstart[blo] // 128) * 128, 0, ncp - w)
    ok = jnp.all(cend[bhi] <= starts + w)
    zeros = jnp.zeros((ntiles,), jnp.int32)
    return lax.cond(
        ok,
        lambda: _topk_select(qp, cp, kout, starts, w, tq),
        lambda: _topk_select(qp, cp, kout, zeros, ncp, tq),
    )


def _sc_gather(table, idx):
    info = plsc.get_sparse_core_info()
    nc, ns = info.num_cores, info.num_subcores
    nw = nc * ns
    b = idx.shape[0]
    d = table.shape[1]
    bpw = b // nw
    chunk = 8
    for c in range(min(bpw, 256), 7, -8):
        if bpw % c == 0:
            chunk = c
            break
    nch = bpw // chunk
    mesh = plsc.VectorSubcoreMesh(core_axis_name="c", subcore_axis_name="s")

    @functools.partial(
        pl.kernel,
        mesh=mesh,
        out_type=jax.ShapeDtypeStruct((b, d), jnp.float32),
        scratch_types=[
            pltpu.VMEM((chunk,), jnp.int32),
            pltpu.VMEM((chunk, d), jnp.float32),
            pltpu.SemaphoreType.DMA,
        ],
    )
    def gk(table_hbm, idx_hbm, out_hbm, idx_v, rows_v, sem):
        wid = lax.axis_index("s") * nc + lax.axis_index("c")
        base = wid * bpw
        for j in range(nch):
            off = base + j * chunk
            pltpu.sync_copy(idx_hbm.at[pl.ds(off, chunk)], idx_v)
            pltpu.async_copy(table_hbm.at[idx_v], rows_v, sem).wait()
            pltpu.sync_copy(rows_v, out_hbm.at[pl.ds(off, chunk)])

    return gk(table, idx)


def _kpconv_kernel(q_ref, d2n_ref, g_ref, kp_ref, w_ref, out_ref, *,
                   r, cin, xoff, tq):
    g = g_ref[...].reshape(tq, NBR, g_ref.shape[1])
    pos_j = g[:, :, 0:3]
    x_j = g[:, :, xoff:xoff + cin]
    qpos = q_ref[...][:, None, 0:3]
    y = pos_j - qpos
    mask = (d2n_ref[...] <= r * r).astype(jnp.float32)
    xn = x_j * mask[:, :, None]
    fs = []
    for k in range(KK):
        kv = kp_ref[k:k + 1, 0:3].reshape(1, 1, 3)
        diff = y - kv
        dist = jnp.sqrt(jnp.sum(diff * diff, axis=2) + 1e-12)
        h = jnp.maximum(0.0, 1.0 - dist / r)
        fs.append(jnp.sum(h[:, :, None] * xn, axis=1))
    f = jnp.concatenate(fs, axis=1)
    kcp = w_ref.shape[0]
    if f.shape[1] < kcp:
        f = jnp.concatenate(
            [f, jnp.zeros((tq, kcp - f.shape[1]), jnp.float32)], axis=1)
    out_ref[...] = jnp.dot(f, w_ref[...], preferred_element_type=jnp.float32)


def _kpconv(qp, d2n, rows, kpts, w_flat, r, cin, xoff, cout, tq=128):
    mqp = qp.shape[0]
    dt = rows.shape[1]
    kcp = w_flat.shape[0]
    grid = (mqp // tq,)
    return pl.pallas_call(
        functools.partial(_kpconv_kernel, r=r, cin=cin, xoff=xoff, tq=tq),
        grid=grid,
        in_specs=[
            pl.BlockSpec((tq, 8), lambda i: (i, 0)),
            pl.BlockSpec((tq, NBR), lambda i: (i, 0)),
            pl.BlockSpec((tq * NBR, dt), lambda i: (i, 0)),
            pl.BlockSpec((16, 8), lambda i: (0, 0)),
            pl.BlockSpec((kcp, cout), lambda i: (0, 0)),
        ],
        out_specs=pl.BlockSpec((tq, cout), lambda i: (i, 0)),
        out_shape=jax.ShapeDtypeStruct((mqp, cout), jnp.float32),
    )(qp, d2n, rows, kpts, w_flat)


def _interp_weights(d2n):
    d = jnp.maximum(d2n, 1e-16)
    w = 1.0 / d
    return w / jnp.sum(w, axis=1, keepdims=True)


def _fp2_kernel(d2n_ref, g_ref, x1_ref, w_ref, b_ref, out_ref, *, tq, c):
    g = g_ref[...].reshape(tq, 3, g_ref.shape[1])[:, :, :c]
    w3 = _interp_weights(d2n_ref[...])
    xi = jnp.sum(g * w3[:, :, None], axis=1)
    cat = jnp.concatenate([xi, x1_ref[...]], axis=1)
    out_ref[...] = (
        jnp.dot(cat, w_ref[...], preferred_element_type=jnp.float32)
        + b_ref[...]
    )


def _fp1_cls_kernel(d2n_ref, g_ref, wa_ref, ba_ref, wb_ref, bb_ref,
                    wc1_ref, bc1_ref, wc2_ref, bc2_ref, out_ref, *, tq, c):
    g = g_ref[...].reshape(tq, 3, g_ref.shape[1])[:, :, :c]
    w3 = _interp_weights(d2n_ref[...])
    xj = jnp.sum(g * w3[:, :, None], axis=1)
    xj = jnp.maximum(
        jnp.dot(xj, wa_ref[...], preferred_element_type=jnp.float32)
        + ba_ref[...], 0.0)
    xj = (jnp.dot(xj, wb_ref[...], preferred_element_type=jnp.float32)
          + bb_ref[...])
    xc = jnp.maximum(
        jnp.dot(xj, wc1_ref[...], preferred_element_type=jnp.float32)
        + bc1_ref[...], 0.0)
    xc = (jnp.dot(xc, wc2_ref[...], preferred_element_type=jnp.float32)
          + bc2_ref[...])
    mx = jnp.max(xc, axis=1, keepdims=True)
    z = xc - mx
    lse = jnp.log(jnp.sum(jnp.exp(z), axis=1, keepdims=True))
    out_ref[...] = z - lse


def _pad_rows(a, n):
    return jnp.pad(a, ((0, n - a.shape[0]), (0, 0)))


def _pack(posn, batchf, pad_to):
    m = posn.shape[0]
    p = jnp.concatenate(
        [posn, batchf[:, None], jnp.zeros((m, 4), jnp.float32)], axis=1)
    if pad_to > m:
        pad = jnp.zeros((pad_to - m, 8), jnp.float32).at[:, 0].set(PAD_COORD)
        p = jnp.concatenate([p, pad], axis=0)
    return p


def kernel(pos, batch, kp1_pts, W1, kp2_pts, W2, W_fp2, b_fp2, W_fp1a,
           b_fp1a, W_fp1b, b_fp1b, Wc1, bc1, Wc2, bc2):
    mx = jnp.max(pos)
    mn = jnp.min(pos)
    posn = (pos - (mx + mn) / 2.0) / jnp.abs(mx - mn)
    bf = batch.astype(jnp.float32)

    m1, m1p = (N_POINTS + 2) // 3, 2816
    m2, m2p = (m1 + 1) // 2, 1408

    p0 = _pack(posn, bf, N_POINTS)
    q1 = _pack(posn[::3], bf[::3], m1p)
    q2 = _pack(posn[::3][::2], bf[::3][::2], m2p)

    b0 = bf
    b1 = bf[::3]
    b2 = bf[::3][::2]
    idx1, d2n1 = _windowed_select(q1, p0, b0, NBR, 4608, tq=256)
    idx2, d2n2 = _windowed_select(q2, q1, b1, NBR, 1664, tq=128)
    idxi2, d2ni2 = _windowed_select(q1, q2, b2, 3, 896, tq=256)
    idxi1, d2ni1 = _windowed_select(p0, q1, b1, 3, 1664, tq=256)

    t1 = jnp.pad(posn, ((0, 0), (0, 125)))
    g1 = _sc_gather(t1, idx1.reshape(-1))
    kp1 = jnp.pad(kp1_pts, ((0, 1), (0, 5)))
    w1f = jnp.pad(W1.reshape(KK * 3, 32), ((0, 3), (0, 0)))
    x1 = _kpconv(q1, d2n1, g1, kp1, w1f, 0.2, 3, 0, 32)

    t2 = jnp.concatenate(
        [q1[:, 0:3], x1, jnp.zeros((m1p, 93), jnp.float32)], axis=1)
    g2 = _sc_gather(t2, idx2.reshape(-1))
    kp2 = jnp.pad(kp2_pts, ((0, 1), (0, 5)))
    w2f = W2.reshape(KK * 32, 64)
    x2 = _kpconv(q2, d2n2, g2, kp2, w2f, 0.4, 32, 3, 64)

    gi2 = _sc_gather(jnp.pad(x2, ((0, 0), (0, 64))),
                     idxi2.reshape(-1))
    tq = 128
    xf = pl.pallas_call(
        functools.partial(_fp2_kernel, tq=tq, c=64),
        grid=(m1p // tq,),
        in_specs=[
            pl.BlockSpec((tq, 3), lambda i: (i, 0)),
            pl.BlockSpec((tq * 3, 128), lambda i: (i, 0)),
            pl.BlockSpec((tq, 32), lambda i: (i, 0)),
            pl.BlockSpec((96, 32), lambda i: (0, 0)),
            pl.BlockSpec((1, 32), lambda i: (0, 0)),
        ],
        out_specs=pl.BlockSpec((tq, 32), lambda i: (i, 0)),
        out_shape=jax.ShapeDtypeStruct((m1p, 32), jnp.float32),
    )(d2ni2, gi2, x1, W_fp2, b_fp2[None, :])

    gi1 = _sc_gather(jnp.pad(xf, ((0, 0), (0, 96))),
                     idxi1.reshape(-1))
    out = pl.pallas_call(
        functools.partial(_fp1_cls_kernel, tq=tq, c=32),
        grid=(N_POINTS // tq,),
        in_specs=[
            pl.BlockSpec((tq, 3), lambda i: (i, 0)),
            pl.BlockSpec((tq * 3, 128), lambda i: (i, 0)),
            pl.BlockSpec((32, 32), lambda i: (0, 0)),
            pl.BlockSpec((1, 32), lambda i: (0, 0)),
            pl.BlockSpec((32, 32), lambda i: (0, 0)),
            pl.BlockSpec((1, 32), lambda i: (0, 0)),
            pl.BlockSpec((32, 16), lambda i: (0, 0)),
            pl.BlockSpec((1, 16), lambda i: (0, 0)),
            pl.BlockSpec((16, 50), lambda i: (0, 0)),
            pl.BlockSpec((1, 50), lambda i: (0, 0)),
        ],
        out_specs=pl.BlockSpec((tq, 50), lambda i: (i, 0)),
        out_shape=jax.ShapeDtypeStruct((N_POINTS, 50), jnp.float32),
    )(d2ni1, gi1, W_fp1a, b_fp1a[None, :], W_fp1b, b_fp1b[None, :],
      Wc1, bc1[None, :], Wc2, bc2[None, :])
    return out
